# Initial kernel scaffold; baseline (speedup 1.0000x reference)
import jax
import jax.numpy as jnp
from jax.experimental import pallas as pl


def kernel(x, edge_index, batch, Wl1, Wr1, att1, b1, Wl2, Wr2, att2, b2, Wfc, bfc):
    raise NotImplementedError("write your pallas kernel here")


# speedup vs baseline: 1.0014x; 1.0014x over previous
import functools

import jax
import jax.numpy as jnp
from jax.experimental import pallas as pl
from jax.experimental.pallas import tpu as pltpu

_N = 10000
_E = 320000
_D = 128
_HID = 128
_HEADS = 8
_OUT = 128
_G = 16

_ROWS_BLK = 1000


def _pool_fc_body(h2_ref, batch_ref, wfc_ref, bfc_ref, logits_ref, prob_ref,
                  sums_ref, cnts_ref):
    i = pl.program_id(0)
    nsteps = pl.num_programs(0)

    @pl.when(i == 0)
    def _init():
        sums_ref[...] = jnp.zeros_like(sums_ref)
        cnts_ref[...] = jnp.zeros_like(cnts_ref)

    h = h2_ref[...]
    b = batch_ref[...]
    gids = jax.lax.broadcasted_iota(jnp.int32, (_G, _ROWS_BLK), 0)
    onehot = jnp.where(gids == b[:, 0][None, :], 1.0, 0.0)
    sums_ref[...] += jnp.dot(onehot, h, preferred_element_type=jnp.float32)
    cnts_ref[...] += jnp.sum(onehot, axis=1, keepdims=True)

    @pl.when(i == nsteps - 1)
    def _fin():
        pooled = sums_ref[...] / jnp.maximum(cnts_ref[...], 1.0)
        logits = jnp.dot(pooled, wfc_ref[...],
                         preferred_element_type=jnp.float32) + bfc_ref[...]
        m = jnp.max(logits, axis=1, keepdims=True)
        e = jnp.exp(logits - m)
        logits_ref[...] = logits
        prob_ref[...] = e / jnp.sum(e, axis=1, keepdims=True)


def _pool_fc(h2, batch, Wfc, bfc):
    batch2 = batch.reshape(_N, 1).astype(jnp.int32)
    bfc2 = bfc.reshape(1, 2)
    grid = _N // _ROWS_BLK
    return pl.pallas_call(
        _pool_fc_body,
        grid=(grid,),
        in_specs=[
            pl.BlockSpec((_ROWS_BLK, _OUT), lambda i: (i, 0)),
            pl.BlockSpec((_ROWS_BLK, 1), lambda i: (i, 0)),
            pl.BlockSpec((_OUT, 2), lambda i: (0, 0)),
            pl.BlockSpec((1, 2), lambda i: (0, 0)),
        ],
        out_specs=[
            pl.BlockSpec((_G, 2), lambda i: (0, 0)),
            pl.BlockSpec((_G, 2), lambda i: (0, 0)),
        ],
        out_shape=[
            jax.ShapeDtypeStruct((_G, 2), jnp.float32),
            jax.ShapeDtypeStruct((_G, 2), jnp.float32),
        ],
        scratch_shapes=[
            pltpu.VMEM((_G, _OUT), jnp.float32),
            pltpu.VMEM((_G, 1), jnp.float32),
        ],
    )(h2, batch2, Wfc, bfc2)


def _gat_layer_xla(x, src, dst, Wl, Wr, att, heads, out_ch):
    n = x.shape[0]
    xl = (x @ Wl).reshape(n, heads, out_ch)
    xr = (x @ Wr).reshape(n, heads, out_ch)
    m = xl[src] + xr[dst]
    ma = jax.nn.leaky_relu(m, negative_slope=0.2)
    alpha = jnp.sum(ma * att[None, :, :], axis=-1)
    amax = jax.ops.segment_max(alpha, dst, num_segments=n)
    amax = jnp.where(jnp.isfinite(amax), amax, 0.0)
    ex = jnp.exp(alpha - amax[dst])
    den = jax.ops.segment_sum(ex, dst, num_segments=n)
    a = ex / (den[dst] + 1e-16)
    out = jax.ops.segment_sum(xl[src] * a[:, :, None], dst, num_segments=n)
    return out


def kernel(x, edge_index, batch, Wl1, Wr1, att1, b1, Wl2, Wr2, att2, b2, Wfc, bfc):
    src, dst = edge_index[0], edge_index[1]
    h = _gat_layer_xla(x, src, dst, Wl1, Wr1, att1, _HEADS, _HID)
    h = h.reshape(-1, _HEADS * _HID) + b1
    h = jax.nn.elu(h)
    h2 = _gat_layer_xla(h, src, dst, Wl2, Wr2, att2, 1, _OUT)
    h2 = h2.reshape(-1, _OUT) + b2
    h2 = jax.nn.elu(h2)
    logits, y_prob = _pool_fc(h2, batch, Wfc, bfc)
    return (logits, y_prob)


# speedup vs baseline: 2.5569x; 2.5534x over previous
import functools

import jax
import jax.numpy as jnp
from jax.experimental import pallas as pl
from jax.experimental.pallas import tpu as pltpu

_N = 10000
_E = 320000
_D = 128
_HID = 128
_HEADS = 8
_OUT = 128
_G = 16


def _mm_body(x_ref, w_ref, o_ref):
    o_ref[...] = jnp.dot(x_ref[...], w_ref[...],
                         preferred_element_type=jnp.float32)


def _mm(x, w, blk_rows):
    n, k = x.shape
    m = w.shape[1]
    return pl.pallas_call(
        _mm_body,
        grid=(n // blk_rows,),
        in_specs=[
            pl.BlockSpec((blk_rows, k), lambda i: (i, 0)),
            pl.BlockSpec((k, m), lambda i: (0, 0)),
        ],
        out_specs=pl.BlockSpec((blk_rows, m), lambda i: (i, 0)),
        out_shape=jax.ShapeDtypeStruct((n, m), jnp.float32),
    )(x, w)


def _mm_elu_body(g_ref, b_ref, w_ref, o_ref):
    h = g_ref[...] + b_ref[...]
    h = jnp.where(h > 0, h, jnp.exp(h) - 1.0)
    o_ref[...] = jnp.dot(h, w_ref[...], preferred_element_type=jnp.float32)


def _mm_elu(g, b, w, blk_rows):
    n, k = g.shape
    m = w.shape[1]
    return pl.pallas_call(
        _mm_elu_body,
        grid=(n // blk_rows,),
        in_specs=[
            pl.BlockSpec((blk_rows, k), lambda i: (i, 0)),
            pl.BlockSpec((1, k), lambda i: (0, 0)),
            pl.BlockSpec((k, m), lambda i: (0, 0)),
        ],
        out_specs=pl.BlockSpec((blk_rows, m), lambda i: (i, 0)),
        out_shape=jax.ShapeDtypeStruct((n, m), jnp.float32),
    )(g, b.reshape(1, k), w)


def _alpha_body(xs_ref, xr_ref, att_ref, kt_ref, o_ref):
    m = xs_ref[...] + xr_ref[...]
    m = jnp.where(m >= 0, m, 0.2 * m)
    o_ref[...] = jnp.dot(m * att_ref[...], kt_ref[...],
                         preferred_element_type=jnp.float32)


def _alpha(xs, xr, att_flat, kt, blk_rows):
    e, w = xs.shape
    h = kt.shape[1]
    return pl.pallas_call(
        _alpha_body,
        grid=(e // blk_rows,),
        in_specs=[
            pl.BlockSpec((blk_rows, w), lambda i: (i, 0)),
            pl.BlockSpec((blk_rows, w), lambda i: (i, 0)),
            pl.BlockSpec((1, w), lambda i: (0, 0)),
            pl.BlockSpec((w, h), lambda i: (0, 0)),
        ],
        out_specs=pl.BlockSpec((blk_rows, h), lambda i: (i, 0)),
        out_shape=jax.ShapeDtypeStruct((e, h), jnp.float32),
    )(xs, xr, att_flat, kt)


_ROWS_BLK = 1000


def _pool_fc_body(h2_ref, b2_ref, batch_ref, wfc_ref, bfc_ref,
                  logits_ref, prob_ref, sums_ref, cnts_ref):
    i = pl.program_id(0)
    nsteps = pl.num_programs(0)

    @pl.when(i == 0)
    def _init():
        sums_ref[...] = jnp.zeros_like(sums_ref)
        cnts_ref[...] = jnp.zeros_like(cnts_ref)

    h = h2_ref[...] + b2_ref[...]
    h = jnp.where(h > 0, h, jnp.exp(h) - 1.0)
    b = batch_ref[...]
    gids = jax.lax.broadcasted_iota(jnp.int32, (_G, _ROWS_BLK), 0)
    onehot = jnp.where(gids == b[:, 0][None, :], 1.0, 0.0)
    sums_ref[...] += jnp.dot(onehot, h, preferred_element_type=jnp.float32)
    cnts_ref[...] += jnp.sum(onehot, axis=1, keepdims=True)

    @pl.when(i == nsteps - 1)
    def _fin():
        pooled = sums_ref[...] / jnp.maximum(cnts_ref[...], 1.0)
        logits = jnp.dot(pooled, wfc_ref[...],
                         preferred_element_type=jnp.float32) + bfc_ref[...]
        mx = jnp.max(logits, axis=1, keepdims=True)
        ex = jnp.exp(logits - mx)
        logits_ref[...] = logits
        prob_ref[...] = ex / jnp.sum(ex, axis=1, keepdims=True)


def _pool_fc(h2raw, b2, batch, Wfc, bfc):
    batch2 = batch.reshape(_N, 1).astype(jnp.int32)
    grid = _N // _ROWS_BLK
    return pl.pallas_call(
        _pool_fc_body,
        grid=(grid,),
        in_specs=[
            pl.BlockSpec((_ROWS_BLK, _OUT), lambda i: (i, 0)),
            pl.BlockSpec((1, _OUT), lambda i: (0, 0)),
            pl.BlockSpec((_ROWS_BLK, 1), lambda i: (i, 0)),
            pl.BlockSpec((_OUT, 2), lambda i: (0, 0)),
            pl.BlockSpec((1, 2), lambda i: (0, 0)),
        ],
        out_specs=[
            pl.BlockSpec((_G, 2), lambda i: (0, 0)),
            pl.BlockSpec((_G, 2), lambda i: (0, 0)),
        ],
        out_shape=[
            jax.ShapeDtypeStruct((_G, 2), jnp.float32),
            jax.ShapeDtypeStruct((_G, 2), jnp.float32),
        ],
        scratch_shapes=[
            pltpu.VMEM((_G, _OUT), jnp.float32),
            pltpu.VMEM((_G, 1), jnp.float32),
        ],
    )(h2raw, b2.reshape(1, _OUT), batch2, Wfc, bfc.reshape(1, 2))


def _segment_softmax_agg_xla(alpha, ds, xs, heads):
    amax = jax.ops.segment_max(alpha, ds, num_segments=_N)
    amax = jnp.where(jnp.isfinite(amax), amax, 0.0)
    ex = jnp.exp(alpha - amax[ds])
    den = jax.ops.segment_sum(ex, ds, num_segments=_N)
    a = ex / (den[ds] + 1e-16)
    a_full = jnp.repeat(a, xs.shape[1] // heads, axis=1)
    return jax.ops.segment_sum(xs * a_full, ds, num_segments=_N)


def kernel(x, edge_index, batch, Wl1, Wr1, att1, b1, Wl2, Wr2, att2, b2, Wfc, bfc):
    src, dst = edge_index[0], edge_index[1]
    perm = jnp.argsort(dst)
    ds = jnp.take(dst, perm)
    ss = jnp.take(src, perm)

    kt8 = (jax.lax.broadcasted_iota(jnp.int32, (_HEADS * _HID, _HEADS), 0)
           // _HID == jax.lax.broadcasted_iota(
               jnp.int32, (_HEADS * _HID, _HEADS), 1)).astype(jnp.float32)

    proj = _mm(x, jnp.concatenate([Wl1, Wr1], axis=1), 1000)
    xl1, xr1 = proj[:, :_HEADS * _HID], proj[:, _HEADS * _HID:]
    xs1 = jnp.take(xl1, ss, axis=0)
    xr1e = jnp.take(xr1, ds, axis=0)
    alpha1 = _alpha(xs1, xr1e, att1.reshape(1, -1), kt8, 1000)
    agg1 = _segment_softmax_agg_xla(alpha1, ds, xs1, _HEADS)

    proj2 = _mm_elu(agg1, b1, jnp.concatenate([Wl2, Wr2], axis=1), 1000)
    hl2, hr2 = proj2[:, :_OUT], proj2[:, _OUT:]
    xs2 = jnp.take(hl2, ss, axis=0)
    xr2e = jnp.take(hr2, ds, axis=0)
    kt1 = jnp.ones((_OUT, 1), jnp.float32)
    alpha2 = _alpha(xs2, xr2e, att2.reshape(1, -1), kt1, 1000)
    agg2 = _segment_softmax_agg_xla(alpha2, ds, xs2, 1)

    logits, y_prob = _pool_fc(agg2, b2, batch, Wfc, bfc)
    return (logits, y_prob)


# speedup vs baseline: 2.6511x; 1.0369x over previous
import functools

import jax
import jax.numpy as jnp
from jax import lax
from jax.experimental import pallas as pl
from jax.experimental.pallas import tpu as pltpu
from jax.experimental.pallas import tpu_sc as plsc

_N = 10000
_E = 320000
_D = 128
_HID = 128
_HEADS = 8
_OUT = 128
_G = 16


def _mm_body(x_ref, w_ref, o_ref):
    o_ref[...] = jnp.dot(x_ref[...], w_ref[...],
                         preferred_element_type=jnp.float32)


def _mm(x, w, blk_rows):
    n, k = x.shape
    m = w.shape[1]
    return pl.pallas_call(
        _mm_body,
        grid=(n // blk_rows,),
        in_specs=[
            pl.BlockSpec((blk_rows, k), lambda i: (i, 0)),
            pl.BlockSpec((k, m), lambda i: (0, 0)),
        ],
        out_specs=pl.BlockSpec((blk_rows, m), lambda i: (i, 0)),
        out_shape=jax.ShapeDtypeStruct((n, m), jnp.float32),
    )(x, w)


def _mm_elu_body(g_ref, b_ref, w_ref, o_ref):
    h = g_ref[...] + b_ref[...]
    h = jnp.where(h > 0, h, jnp.exp(h) - 1.0)
    o_ref[...] = jnp.dot(h, w_ref[...], preferred_element_type=jnp.float32)


def _mm_elu(g, b, w, blk_rows):
    n, k = g.shape
    m = w.shape[1]
    return pl.pallas_call(
        _mm_elu_body,
        grid=(n // blk_rows,),
        in_specs=[
            pl.BlockSpec((blk_rows, k), lambda i: (i, 0)),
            pl.BlockSpec((1, k), lambda i: (0, 0)),
            pl.BlockSpec((k, m), lambda i: (0, 0)),
        ],
        out_specs=pl.BlockSpec((blk_rows, m), lambda i: (i, 0)),
        out_shape=jax.ShapeDtypeStruct((n, m), jnp.float32),
    )(g, b.reshape(1, k), w)


def _alpha_body(xs_ref, xr_ref, att_ref, kt_ref, o_ref):
    m = xs_ref[...] + xr_ref[...]
    m = jnp.where(m >= 0, m, 0.2 * m)
    o_ref[...] = jnp.dot(m * att_ref[...], kt_ref[...],
                         preferred_element_type=jnp.float32)


def _alpha(xs, xr, att_flat, kt, blk_rows):
    e, w = xs.shape
    h = kt.shape[1]
    return pl.pallas_call(
        _alpha_body,
        grid=(e // blk_rows,),
        in_specs=[
            pl.BlockSpec((blk_rows, w), lambda i: (i, 0)),
            pl.BlockSpec((blk_rows, w), lambda i: (i, 0)),
            pl.BlockSpec((1, w), lambda i: (0, 0)),
            pl.BlockSpec((w, h), lambda i: (0, 0)),
        ],
        out_specs=pl.BlockSpec((blk_rows, h), lambda i: (i, 0)),
        out_shape=jax.ShapeDtypeStruct((e, h), jnp.float32),
    )(xs, xr, att_flat, kt)


_ROWS_BLK = 1000


def _pool_fc_body(h2_ref, b2_ref, batch_ref, wfc_ref, bfc_ref,
                  logits_ref, prob_ref, sums_ref, cnts_ref):
    i = pl.program_id(0)
    nsteps = pl.num_programs(0)

    @pl.when(i == 0)
    def _init():
        sums_ref[...] = jnp.zeros_like(sums_ref)
        cnts_ref[...] = jnp.zeros_like(cnts_ref)

    h = h2_ref[...] + b2_ref[...]
    h = jnp.where(h > 0, h, jnp.exp(h) - 1.0)
    b = batch_ref[...]
    gids = jax.lax.broadcasted_iota(jnp.int32, (_G, _ROWS_BLK), 0)
    onehot = jnp.where(gids == b[:, 0][None, :], 1.0, 0.0)
    sums_ref[...] += jnp.dot(onehot, h, preferred_element_type=jnp.float32)
    cnts_ref[...] += jnp.sum(onehot, axis=1, keepdims=True)

    @pl.when(i == nsteps - 1)
    def _fin():
        pooled = sums_ref[...] / jnp.maximum(cnts_ref[...], 1.0)
        logits = jnp.dot(pooled, wfc_ref[...],
                         preferred_element_type=jnp.float32) + bfc_ref[...]
        mx = jnp.max(logits, axis=1, keepdims=True)
        ex = jnp.exp(logits - mx)
        logits_ref[...] = logits
        prob_ref[...] = ex / jnp.sum(ex, axis=1, keepdims=True)


def _pool_fc(h2raw, b2, batch, Wfc, bfc):
    batch2 = batch.reshape(_N, 1).astype(jnp.int32)
    grid = _N // _ROWS_BLK
    return pl.pallas_call(
        _pool_fc_body,
        grid=(grid,),
        in_specs=[
            pl.BlockSpec((_ROWS_BLK, _OUT), lambda i: (i, 0)),
            pl.BlockSpec((1, _OUT), lambda i: (0, 0)),
            pl.BlockSpec((_ROWS_BLK, 1), lambda i: (i, 0)),
            pl.BlockSpec((_OUT, 2), lambda i: (0, 0)),
            pl.BlockSpec((1, 2), lambda i: (0, 0)),
        ],
        out_specs=[
            pl.BlockSpec((_G, 2), lambda i: (0, 0)),
            pl.BlockSpec((_G, 2), lambda i: (0, 0)),
        ],
        out_shape=[
            jax.ShapeDtypeStruct((_G, 2), jnp.float32),
            jax.ShapeDtypeStruct((_G, 2), jnp.float32),
        ],
        scratch_shapes=[
            pltpu.VMEM((_G, _OUT), jnp.float32),
            pltpu.VMEM((_G, 1), jnp.float32),
        ],
    )(h2raw, b2.reshape(1, _OUT), batch2, Wfc, bfc.reshape(1, 2))


_NC = 2
_NW = 32
_NPW = 320
_NPAD = _NW * _NPW
_RSLEN = 336
_RSTOT = 31 * _NPW + _RSLEN
_SUB = 64
_ROUNDS = _NPW // _SUB
_EPW = _E // _NW
_ACH = 80
_BIG = 3.0e38


def _sc_mesh():
    return plsc.VectorSubcoreMesh(core_axis_name="c", subcore_axis_name="s")


def _wid():
    return lax.axis_index("s") * _NC + lax.axis_index("c")


def _iota16():
    return lax.iota(jnp.int32, 16)


def _splat_i(v):
    return jnp.full((16,), v, jnp.int32)


def _sc_gather(xl, xr, ss, ds, w):
    @functools.partial(
        pl.kernel,
        mesh=_sc_mesh(),
        out_type=(
            jax.ShapeDtypeStruct((_E, w), jnp.float32),
            jax.ShapeDtypeStruct((_E, w), jnp.float32),
        ),
        scratch_types=[
            pltpu.VMEM((_ACH,), jnp.int32),
            pltpu.VMEM((_ACH, w), jnp.float32),
            pltpu.SemaphoreType.DMA,
        ],
    )
    def k(xl_h, xr_h, ss_h, ds_h, xs_h, xrr_h, idx_v, rows_v, sem):
        wid = _wid()

        def phase(src_idx_h, table_h, out_h):
            def body(i, c):
                base = wid * _EPW + i * _ACH
                pltpu.sync_copy(src_idx_h.at[pl.ds(base, _ACH)], idx_v)
                pltpu.async_copy(table_h.at[idx_v], rows_v, sem).wait()
                pltpu.sync_copy(rows_v, out_h.at[pl.ds(base, _ACH)])
                return c
            lax.fori_loop(0, _EPW // _ACH, body, 0)

        phase(ss_h, xl_h, xs_h)
        phase(ds_h, xr_h, xrr_h)

    return k(xl, xr, ss, ds)


def _seg_scan_setup(kbuf_v, kseg):
    kbuf_v[pl.ds(16, 16)] = kseg
    knext = kbuf_v[pl.ds(17, 16)]
    return knext


def _seg_scan(vbuf_v, kbuf_v, kseg, v, is_max):
    for k in (1, 2, 4, 8):
        vbuf_v[pl.ds(16, 16)] = v
        sv = vbuf_v[pl.ds(16 - k, 16)]
        sk = kbuf_v[pl.ds(16 - k, 16)]
        comb = jnp.maximum(v, sv) if is_max else v + sv
        v = jnp.where(sk == kseg, comb, v)
    return v


def _sc_stats(alpha_flat, ds, rs, h):
    @functools.partial(
        pl.kernel,
        mesh=_sc_mesh(),
        compiler_params=pltpu.CompilerParams(needs_layout_passes=False),
        out_type=jax.ShapeDtypeStruct((_NPAD * 16,), jnp.float32),
        scratch_types=[
            pltpu.VMEM((_RSLEN,), jnp.int32),
            pltpu.VMEM((16 * h,), jnp.float32),
            pltpu.VMEM((16,), jnp.int32),
            pltpu.VMEM((48,), jnp.float32),
            pltpu.VMEM((48,), jnp.int32),
            pltpu.VMEM((_NPW * 16,), jnp.float32),
            pltpu.SemaphoreType.DMA,
        ],
    )
    def k(alpha_h, ds_h, rs_h, stats_h, rs_v, ach_v, ds_v, vbuf_v, kbuf_v,
          loc_v, sem):
        wid = _wid()
        node0 = wid * _NPW
        pltpu.sync_copy(rs_h.at[pl.ds(node0, _RSLEN)], rs_v)
        e_lo = rs_v[pl.ds(0, 16)][0]
        e_hi = rs_v[pl.ds(_NPW, 16)][0]
        base0 = (e_lo // 16) * 16
        niters = (e_hi - base0 + 15) // 16

        def init_body(i, c):
            loc_v[pl.ds(i * 16, 16)] = jnp.where(_iota16() < 8, -_BIG, 0.0)
            return c
        lax.fori_loop(0, _NPW, init_body, 0)

        kbuf_v[pl.ds(0, 16)] = _splat_i(-7)
        kbuf_v[pl.ds(32, 16)] = _splat_i(-8)

        def sweep(is_max):
            vbuf_v[pl.ds(0, 16)] = jnp.full((16,), -_BIG if is_max else 0.0,
                                            jnp.float32)

            def body(i, c):
                e0 = base0 + i * 16
                pltpu.sync_copy(alpha_h.at[pl.ds(e0 * h, 16 * h)], ach_v)
                pltpu.sync_copy(ds_h.at[pl.ds(e0, 16)], ds_v)
                eidx = _splat_i(e0) + _iota16()
                valid = (eidx >= e_lo) & (eidx < e_hi)
                dloc = ds_v[...] - node0
                kseg = jnp.where(valid, dloc, -5)
                kidx = jnp.clip(dloc, 0, _NPW - 1) * 16
                knext = _seg_scan_setup(kbuf_v, kseg)
                last = (kseg != knext) & valid
                for hh in range(h):
                    if h > 1:
                        v = plsc.load_gather(ach_v, [_iota16() * h + hh])
                    else:
                        v = ach_v[...]
                    if is_max:
                        v = jnp.where(valid, v, -_BIG)
                    else:
                        am = plsc.load_gather(loc_v, [kidx + hh], mask=valid)
                        v = jnp.where(valid, jnp.exp(v - am), 0.0)
                    v = _seg_scan(vbuf_v, kbuf_v, kseg, v, is_max)
                    col = kidx + (hh if is_max else 8 + hh)
                    cur = plsc.load_gather(loc_v, [col], mask=last)
                    upd = jnp.maximum(cur, v) if is_max else cur + v
                    plsc.store_scatter(loc_v, [col], upd, mask=last)
                return c
            lax.fori_loop(0, niters, body, 0)

        sweep(True)
        sweep(False)
        pltpu.sync_copy(loc_v, stats_h.at[pl.ds(node0 * 16, _NPW * 16)])

    return k(alpha_flat, ds, rs)


def _sc_agg(xs_flat, alpha_flat, ds, rs, stats_flat, w, h):
    jrange = w // 128

    @functools.partial(
        pl.kernel,
        mesh=_sc_mesh(),
        compiler_params=pltpu.CompilerParams(needs_layout_passes=False),
        out_type=jax.ShapeDtypeStruct((_NPAD * w,), jnp.float32),
        scratch_types=[
            pltpu.VMEM((_RSLEN,), jnp.int32),
            pltpu.VMEM((16 * w,), jnp.float32),
            pltpu.VMEM((_SUB * w,), jnp.float32),
            pltpu.VMEM((16 * h,), jnp.float32),
            pltpu.VMEM((16,), jnp.int32),
            pltpu.VMEM((_NPW * 16,), jnp.float32),
            pltpu.VMEM((16 * h,), jnp.float32),
            pltpu.SemaphoreType.DMA,
        ],
    )
    def k(xs_h, alpha_h, ds_h, rs_h, stats_h, agg_h,
          rs_v, rows_v, acc_v, ach_v, ds_v, st_v, cb_v, sem):
        wid = _wid()
        node0 = wid * _NPW
        pltpu.sync_copy(rs_h.at[pl.ds(node0, _RSLEN)], rs_v)
        pltpu.sync_copy(stats_h.at[pl.ds(node0 * 16, _NPW * 16)], st_v)

        def round_body(r, cr):
            nlo = r * _SUB
            e_lo = rs_v[pl.ds(nlo, 16)][0]
            e_hi = rs_v[pl.ds(nlo + _SUB, 16)][0]
            base0 = (e_lo // 16) * 16
            niters = (e_hi - base0 + 15) // 16

            def zero_body(i, c):
                acc_v[pl.ds(i * 16, 16)] = jnp.zeros((16,), jnp.float32)
                return c
            lax.fori_loop(0, _SUB * w // 16, zero_body, 0)

            def body(i, c):
                e0 = base0 + i * 16
                pltpu.sync_copy(xs_h.at[pl.ds(e0 * w, 16 * w)], rows_v)
                pltpu.sync_copy(ds_h.at[pl.ds(e0, 16)], ds_v)
                pltpu.sync_copy(alpha_h.at[pl.ds(e0 * h, 16 * h)], ach_v)
                eidx = _splat_i(e0) + _iota16()
                valid = (eidx >= e_lo) & (eidx < e_hi)
                dall = ds_v[...] - node0
                srow = jnp.clip(dall, 0, _NPW - 1) * 16
                dloc = jnp.clip(dall - nlo, 0, _SUB - 1)
                for hh in range(h):
                    if h > 1:
                        av = plsc.load_gather(ach_v, [_iota16() * h + hh])
                    else:
                        av = ach_v[...]
                    am = plsc.load_gather(st_v, [srow + hh])
                    dn = plsc.load_gather(st_v, [srow + 8 + hh])
                    coef = jnp.exp(av - am) / (dn + 1e-16)
                    coef = jnp.where(valid, coef, 0.0)
                    if h > 1:
                        plsc.store_scatter(cb_v, [_iota16() * h + hh], coef)
                    else:
                        cb_v[...] = coef
                for e in range(16):
                    base = dloc[e] * w

                    def jbody(j, c2, e=e):
                        cf = plsc.load_gather(
                            cb_v, [_splat_i(e * h) + (j if h > 1 else 0)])
                        for u in range(8):
                            off = j * 128 + u * 16
                            acc_v[pl.ds(base + off, 16)] += (
                                cf * rows_v[pl.ds(e * w + off, 16)])
                        return c2
                    lax.fori_loop(0, jrange, jbody, 0)
                return c
            lax.fori_loop(0, niters, body, 0)
            pltpu.sync_copy(
                acc_v, agg_h.at[pl.ds((node0 + nlo) * w, _SUB * w)])
            return cr
        lax.fori_loop(0, _ROUNDS, round_body, 0)

    return k(xs_flat, alpha_flat, ds, rs, stats_flat)


def _segment_softmax_agg_sc(alpha, ds, rs, xs, heads):
    w = xs.shape[1]
    af = alpha.reshape(-1)
    stats = _sc_stats(af, ds, rs, heads)
    if True:
        st = stats.reshape(_NPAD, 16)
        amax = st[:_N, :heads]
        den = st[:_N, 8:8 + heads]
        a = jnp.exp(alpha - amax[ds]) / (den[ds] + 1e-16)
        a_full = jnp.repeat(a, w // heads, axis=1)
        return jax.ops.segment_sum(xs * a_full, ds, num_segments=_N)
    agg = _sc_agg(xs.reshape(-1), af, ds, rs, stats, w, heads)
    return agg.reshape(_NPAD, w)[:_N]


def _segment_softmax_agg_xla(alpha, ds, xs, heads):
    amax = jax.ops.segment_max(alpha, ds, num_segments=_N)
    amax = jnp.where(jnp.isfinite(amax), amax, 0.0)
    ex = jnp.exp(alpha - amax[ds])
    den = jax.ops.segment_sum(ex, ds, num_segments=_N)
    a = ex / (den[ds] + 1e-16)
    a_full = jnp.repeat(a, xs.shape[1] // heads, axis=1)
    return jax.ops.segment_sum(xs * a_full, ds, num_segments=_N)


def kernel(x, edge_index, batch, Wl1, Wr1, att1, b1, Wl2, Wr2, att2, b2, Wfc, bfc):
    src, dst = edge_index[0], edge_index[1]
    perm = jnp.argsort(dst)
    ds = jnp.take(dst, perm)
    ss = jnp.take(src, perm)
    rs = jnp.searchsorted(ds, jnp.arange(_RSTOT, dtype=jnp.int32)
                          ).astype(jnp.int32)

    kt8 = (jax.lax.broadcasted_iota(jnp.int32, (_HEADS * _HID, _HEADS), 0)
           // _HID == jax.lax.broadcasted_iota(
               jnp.int32, (_HEADS * _HID, _HEADS), 1)).astype(jnp.float32)

    proj = _mm(x, jnp.concatenate([Wl1, Wr1], axis=1), 1000)
    xl1 = proj[:, :_HEADS * _HID]
    xr1 = proj[:, _HEADS * _HID:]
    xs1, xr1e = _sc_gather(xl1, xr1, ss, ds, _HEADS * _HID)
    alpha1 = _alpha(xs1, xr1e, att1.reshape(1, -1), kt8, 1000)
    agg1 = _segment_softmax_agg_sc(alpha1, ds, rs, xs1, _HEADS)

    proj2 = _mm_elu(agg1, b1, jnp.concatenate([Wl2, Wr2], axis=1), 1000)
    hl2 = proj2[:, :_OUT]
    hr2 = proj2[:, _OUT:]
    xs2, xr2e = _sc_gather(hl2, hr2, ss, ds, _OUT)
    kt1 = jnp.ones((_OUT, 1), jnp.float32)
    alpha2 = _alpha(xs2, xr2e, att2.reshape(1, -1), kt1, 1000)
    agg2 = _segment_softmax_agg_sc(alpha2, ds, rs, xs2, 1)

    logits, y_prob = _pool_fc(agg2, b2, batch, Wfc, bfc)
    return (logits, y_prob)
